# Initial kernel scaffold; baseline (speedup 1.0000x reference)
import jax
import jax.numpy as jnp
from jax.experimental import pallas as pl


def kernel(i, PE):
    raise NotImplementedError("write your pallas kernel here")


# speedup vs baseline: 6.0431x; 6.0431x over previous
import functools

import jax
import jax.numpy as jnp
from jax import lax
from jax.experimental import pallas as pl
from jax.experimental.pallas import tpu as pltpu
from jax.experimental.pallas import tpu_sc as plsc

CHUNK = 1024


@functools.lru_cache(maxsize=None)
def _build(B: int, V: int, D: int):
    info = plsc.get_sparse_core_info()
    NC, NS = info.num_cores, info.num_subcores
    NW = NC * NS
    assert B % NW == 0
    b_per_w = B // NW
    assert b_per_w % CHUNK == 0
    nch = b_per_w // CHUNK

    mesh = plsc.VectorSubcoreMesh(core_axis_name="c", subcore_axis_name="s")

    @functools.partial(
        pl.kernel,
        mesh=mesh,
        out_type=jax.ShapeDtypeStruct((B, D), jnp.float32),
        scratch_types=[
            pltpu.VMEM((CHUNK,), jnp.int32),
            pltpu.VMEM((CHUNK, D), jnp.float32),
            pltpu.SemaphoreType.DMA,
        ],
        compiler_params=pltpu.CompilerParams(use_tc_tiling_on_sc=False),
    )
    def gather_kernel(idx_hbm, table_hbm, out_hbm, idx_v, rows_v, sem):
        wid = lax.axis_index("s") * NC + lax.axis_index("c")
        base = wid * b_per_w

        def body(ch, _):
            off = base + ch * CHUNK
            pltpu.sync_copy(idx_hbm.at[pl.ds(off, CHUNK)], idx_v)
            pltpu.async_copy(table_hbm.at[idx_v], rows_v, sem).wait()
            pltpu.sync_copy(rows_v, out_hbm.at[pl.ds(off, CHUNK)])
            return 0

        lax.fori_loop(0, nch, body, 0)

    return gather_kernel


def kernel(i, PE):
    V, D = PE.shape
    B = i.size
    iflat = i.reshape(B).astype(jnp.int32)
    out = _build(B, V, D)(iflat, PE)
    return out.reshape(i.shape + (D,))


# speedup vs baseline: 6.5274x; 1.0801x over previous
import functools

import jax
import jax.numpy as jnp
from jax import lax
from jax.experimental import pallas as pl
from jax.experimental.pallas import tpu as pltpu
from jax.experimental.pallas import tpu_sc as plsc

CHUNK = 1024
NB = 4


@functools.lru_cache(maxsize=None)
def _build(B: int, V: int, D: int):
    info = plsc.get_sparse_core_info()
    NC, NS = info.num_cores, info.num_subcores
    NW = NC * NS
    assert B % NW == 0
    b_per_w = B // NW
    assert b_per_w % (CHUNK * NB) == 0
    nch = b_per_w // CHUNK

    mesh = plsc.VectorSubcoreMesh(core_axis_name="c", subcore_axis_name="s")

    @functools.partial(
        pl.kernel,
        mesh=mesh,
        out_type=jax.ShapeDtypeStruct((B, D), jnp.float32),
        scratch_types=[
            pltpu.VMEM((NB, CHUNK), jnp.int32),
            pltpu.VMEM((NB, CHUNK, D), jnp.float32),
            pltpu.SemaphoreType.DMA((NB,)),
            pltpu.SemaphoreType.DMA((NB,)),
            pltpu.SemaphoreType.DMA((NB,)),
        ],
        compiler_params=pltpu.CompilerParams(use_tc_tiling_on_sc=False),
    )
    def gather_kernel(idx_hbm, table_hbm, out_hbm, idx_v, rows_v, si, sg, so):
        wid = lax.axis_index("s") * NC + lax.axis_index("c")
        base = wid * b_per_w

        def start_idx(ch, b):
            off = base + ch * CHUNK
            pltpu.async_copy(idx_hbm.at[pl.ds(off, CHUNK)], idx_v.at[b],
                             si.at[b])

        def start_gather(b):
            pltpu.async_copy(table_hbm.at[idx_v.at[b]], rows_v.at[b], sg.at[b])

        def start_out(ch, b):
            off = base + ch * CHUNK
            pltpu.async_copy(rows_v.at[b], out_hbm.at[pl.ds(off, CHUNK)],
                             so.at[b])

        for b in range(NB):
            start_idx(b, b)
        for b in range(NB):
            pltpu.make_async_copy(idx_hbm.at[pl.ds(base, CHUNK)],
                                  idx_v.at[b], si.at[b]).wait()
            start_gather(b)

        @pl.loop(0, nch, step=NB)
        def round_(g):
            for b in range(NB):
                ch = g + b
                pltpu.make_async_copy(table_hbm.at[idx_v.at[b]],
                                      rows_v.at[b], sg.at[b]).wait()
                start_out(ch, b)

                nxt = ch + NB

                @pl.when(nxt < nch)
                def _():
                    start_idx(nxt, b)

            for b in range(NB):
                ch = g + b
                nxt = ch + NB

                @pl.when(nxt < nch)
                def _():
                    pltpu.make_async_copy(rows_v.at[b],
                                          out_hbm.at[pl.ds(base, CHUNK)],
                                          so.at[b]).wait()
                    pltpu.make_async_copy(idx_hbm.at[pl.ds(base, CHUNK)],
                                          idx_v.at[b], si.at[b]).wait()
                    start_gather(b)

        for b in range(NB):
            pltpu.make_async_copy(rows_v.at[b],
                                  out_hbm.at[pl.ds(base, CHUNK)],
                                  so.at[b]).wait()

    return gather_kernel


def kernel(i, PE):
    V, D = PE.shape
    B = i.size
    iflat = i.reshape(B).astype(jnp.int32)
    out = _build(B, V, D)(iflat, PE)
    return out.reshape(i.shape + (D,))
